# baseline (speedup 1.0000x reference)
import functools

import jax
import jax.numpy as jnp
from jax import lax
from jax.experimental import pallas as pl
from jax.experimental.pallas import tpu as pltpu
from jax.experimental.pallas import tpu_sc as plsc

B = 16384
F = 26
ND = 13
NC = 2
NS = 16
L = 16
NW = NC * NS
BPW = B // NW
CH = 128
NCHUNK = (BPW * F) // CH
NWAVE = 2
CPW = NCHUNK // NWAVE
FPW = F // NWAVE
KFIRE = 4
RL = 8
VTAB = 1000000

_mesh = plsc.VectorSubcoreMesh(core_axis_name="c", subcore_axis_name="s")


@functools.partial(
    pl.kernel,
    out_type=jax.ShapeDtypeStruct((B,), jnp.float32),
    mesh=_mesh,
    scratch_types=[
        pltpu.VMEM((NCHUNK, CH), jnp.int32),
        pltpu.VMEM((NCHUNK, CH), jnp.int32),
        pltpu.VMEM((CPW * CH, RL), jnp.float32),
        pltpu.VMEM((ND, BPW), jnp.float32),
        pltpu.VMEM((ND + 2, L), jnp.float32),
        pltpu.VMEM((BPW,), jnp.float32),
        pltpu.SemaphoreType.DMA,
    ],
    compiler_params=pltpu.CompilerParams(
        needs_layout_passes=False, use_tc_tiling_on_sc=False
    ),
)
def _lr_kernel(table_hbm, idxr_hbm, idxc_hbm, dense_hbm, w_hbm, out_hbm,
               idxr_v, idxc_v, vals_v, dense_v, w_v, acc_v, sem):
    w = lax.axis_index("s") * NC + lax.axis_index("c")

    pltpu.sync_copy(idxr_hbm.at[w], idxr_v)
    pltpu.sync_copy(idxc_hbm.at[w], idxc_v)
    pltpu.sync_copy(dense_hbm.at[w], dense_v)
    pltpu.sync_copy(w_hbm, w_v)

    lane = lax.iota(jnp.int32, L)
    rpc = CH // L

    def wave(h, init):
        def fire_group(g, carry):
            for t in range(KFIRE):
                j = g * KFIRE + t
                pltpu.async_copy(
                    table_hbm.at[idxr_v.at[h * CPW + j]],
                    vals_v.at[pl.ds(j * CH, CH)],
                    sem,
                )
            return carry

        lax.fori_loop(0, CPW // KFIRE, fire_group, 0)

        def drain(j, carry):
            pltpu.make_async_copy(
                table_hbm.at[idxr_v.at[h * CPW + j]],
                vals_v.at[pl.ds(j * CH, CH)],
                sem,
            ).wait()
            return carry

        lax.fori_loop(0, CPW, drain, 0)

        def row_block(i, carry):
            ro = i // rpc
            co = (i % rpc) * L
            a = acc_v[pl.ds(i * L, L)] if init else None
            for fl in range(FPW):
                cr = h * CPW + fl * (BPW // CH) + ro
                col16 = idxc_v[cr, pl.ds(co, L)]
                pos16 = lane + (fl * BPW + i * L)
                g16 = plsc.load_gather(vals_v, [pos16, col16])
                a = g16 if a is None else a + g16
            acc_v[pl.ds(i * L, L)] = a
            return carry

        lax.fori_loop(0, BPW // L, row_block, 0)
        return init

    wave(0, False)
    wave(1, True)

    bvec = w_v[ND] + w_v[ND + 1]

    def dense_block(i, carry):
        dsl = pl.ds(i * L, L)
        a = acc_v[dsl] + bvec
        for d in range(ND):
            a = a + dense_v[d, dsl] * w_v[d]
        acc_v[dsl] = a
        return carry

    lax.fori_loop(0, BPW // L, dense_block, 0)

    pltpu.sync_copy(acc_v, out_hbm.at[pl.ds(w * BPW, BPW)])


def kernel(dense, sparse, sparse_table, dense_W, dense_b, bias):
    idx = (
        sparse.astype(jnp.int32)
        .reshape(NW, BPW, F)
        .transpose(0, 2, 1)
    )
    idxr = idx.reshape(NW, NCHUNK, CH)
    idxc = (idx * 0).reshape(NW, NCHUNK, CH)
    table8 = jnp.pad(sparse_table, ((0, 0), (0, RL - 1)))
    dense_prep = dense.reshape(NW, BPW, ND).transpose(0, 2, 1)
    w_prep = jnp.concatenate(
        [
            jnp.broadcast_to(dense_W.reshape(ND, 1), (ND, L)),
            jnp.broadcast_to(dense_b.reshape(1, 1), (1, L)),
            jnp.broadcast_to(bias.reshape(1, 1), (1, L)),
        ],
        axis=0,
    )
    return _lr_kernel(table8, idxr, idxc, dense_prep, w_prep)


# speedup vs baseline: 10.0735x; 10.0735x over previous
import functools

import jax
import jax.numpy as jnp
from jax import lax
from jax.experimental import pallas as pl
from jax.experimental.pallas import tpu as pltpu
from jax.experimental.pallas import tpu_sc as plsc

B = 16384
F = 26
ND = 13
NC = 2
NS = 16
L = 16
NW = NC * NS
BPW = B // NW
CH = 128
NCHUNK = (BPW * F) // CH
KFIRE = 8
VTAB = 1000000

_mesh = plsc.VectorSubcoreMesh(core_axis_name="c", subcore_axis_name="s")


@functools.partial(
    pl.kernel,
    out_type=jax.ShapeDtypeStruct((B,), jnp.float32),
    mesh=_mesh,
    scratch_types=[
        pltpu.VMEM((NCHUNK, CH), jnp.int32),
        pltpu.VMEM((NCHUNK, CH), jnp.float32),
        pltpu.VMEM((ND, BPW), jnp.float32),
        pltpu.VMEM((ND + 2, L), jnp.float32),
        pltpu.VMEM((BPW,), jnp.float32),
        pltpu.SemaphoreType.DMA,
    ],
)
def _lr_kernel(table_hbm, idx_hbm, dense_hbm, w_hbm, out_hbm,
               idx_v, vals_v, dense_v, w_v, acc_v, sem):
    w = lax.axis_index("s") * NC + lax.axis_index("c")

    pltpu.sync_copy(idx_hbm.at[w], idx_v)
    pltpu.sync_copy(dense_hbm.at[w], dense_v)
    pltpu.sync_copy(w_hbm, w_v)

    def fire_group(g, carry):
        for t in range(KFIRE):
            pltpu.async_copy(
                table_hbm.at[idx_v.at[g * KFIRE + t]],
                vals_v.at[g * KFIRE + t],
                sem,
            )
        return carry

    lax.fori_loop(0, NCHUNK // KFIRE, fire_group, 0)

    def drain(j, carry):
        pltpu.make_async_copy(
            table_hbm.at[idx_v.at[j]], vals_v.at[j], sem
        ).wait()
        return carry

    lax.fori_loop(0, NCHUNK, drain, 0)

    bvec = w_v[ND] + w_v[ND + 1]
    rpc = CH // L

    def row_block(i, carry):
        ro = i // rpc
        c = (i % rpc) * L
        a = bvec
        for f in range(F):
            a = a + vals_v[f * (BPW // CH) + ro, pl.ds(c, L)]
        dsl = pl.ds(i * L, L)
        for d in range(ND):
            a = a + dense_v[d, dsl] * w_v[d]
        acc_v[dsl] = a
        return carry

    lax.fori_loop(0, BPW // L, row_block, 0)

    pltpu.sync_copy(acc_v, out_hbm.at[pl.ds(w * BPW, BPW)])


def kernel(dense, sparse, sparse_table, dense_W, dense_b, bias):
    idx = (
        sparse.astype(jnp.int32)
        .reshape(NW, BPW, F)
        .transpose(0, 2, 1)
        .reshape(NW, NCHUNK, CH)
    )
    dense_prep = dense.reshape(NW, BPW, ND).transpose(0, 2, 1)
    w_prep = jnp.concatenate(
        [
            jnp.broadcast_to(dense_W.reshape(ND, 1), (ND, L)),
            jnp.broadcast_to(dense_b.reshape(1, 1), (1, L)),
            jnp.broadcast_to(bias.reshape(1, 1), (1, L)),
        ],
        axis=0,
    )
    table_flat = sparse_table.reshape(VTAB)
    return _lr_kernel(table_flat, idx, dense_prep, w_prep)
